# Initial kernel scaffold; baseline (speedup 1.0000x reference)
import jax
import jax.numpy as jnp
from jax.experimental import pallas as pl


def kernel(node_feature, edge_index, edge_feature, W):
    raise NotImplementedError("write your pallas kernel here")


# speedup vs baseline: 1.6530x; 1.6530x over previous
import functools

import jax
import jax.numpy as jnp
from jax import lax
from jax.experimental import pallas as pl
from jax.experimental.pallas import tpu as pltpu
from jax.experimental.pallas import tpu_sc as plsc

N_NODES = 10000
N_EDGES = 320000
D_FEAT = 128
D_EDGE = 16
D_OUT = 128

NC = 2
NS = 16
NW = NC * NS
E_PER_W = N_EDGES // NW
CHUNK = 80
GRP = 5
NGRP = E_PER_W // (CHUNK * GRP)
N_PAD = 10240
NROWCH = N_PAD // CHUNK


def _sc_segsum_kernel(node_hbm, src_hbm, dst_hbm, ef_hbm, ridx_hbm,
                      sx_out, se_out,
                      sidx, didx, ridx_v, rows_v, ef_v, ef_w, acc_x, acc_e,
                      sem):
    c = lax.axis_index("c")
    s = lax.axis_index("s")
    t = c * NS + s

    @pl.loop(0, CHUNK)
    def zrow(i):
        for k in range(D_FEAT // 16):
            rows_v[i, pl.ds(k * 16, 16)] = jnp.zeros((16,), jnp.float32)
        ef_v[i, :] = jnp.zeros((16,), jnp.float32)

    pltpu.sync_copy(ridx_hbm.at[s], ridx_v)
    @pl.loop(0, NROWCH // NS)
    def zinit(k):
        pltpu.sync_copy(rows_v, acc_x.at[ridx_v.at[k]])
        pltpu.sync_copy(ef_v, acc_e.at[ridx_v.at[k]])

    plsc.subcore_barrier()

    @pl.loop(0, NGRP)
    def group(g):
        pltpu.sync_copy(src_hbm.at[t, g], sidx)
        pltpu.sync_copy(dst_hbm.at[t, g], didx)
        for jj in range(GRP):
            pltpu.async_copy(node_hbm.at[sidx.at[jj]], rows_v, sem).wait()
            pltpu.sync_copy(rows_v, acc_x.at[didx.at[jj]], add=True)

    plsc.subcore_barrier()

    @pl.loop(0, NROWCH // NS)
    def extract(k):
        i = s + NS * k
        pltpu.sync_copy(acc_x.at[ridx_v.at[k]], rows_v)
        pltpu.sync_copy(rows_v, sx_out.at[c, i])
        pltpu.sync_copy(acc_e.at[ridx_v.at[k]], ef_v)
        pltpu.sync_copy(ef_v, se_out.at[c, i])


_sc_segsum = functools.partial(
    pl.kernel,
    out_type=(
        jax.ShapeDtypeStruct((NC, NROWCH, CHUNK, D_FEAT), jnp.float32),
        jax.ShapeDtypeStruct((NC, NROWCH, CHUNK, D_EDGE), jnp.float32),
    ),
    mesh=plsc.VectorSubcoreMesh(core_axis_name="c", subcore_axis_name="s",
                                num_cores=NC, num_subcores=NS),
    scratch_types=[
        pltpu.VMEM((GRP, CHUNK), jnp.int32),
        pltpu.VMEM((GRP, CHUNK), jnp.int32),
        pltpu.VMEM((NROWCH // NS, CHUNK), jnp.int32),
        pltpu.VMEM((CHUNK, D_FEAT), jnp.float32),
        pltpu.VMEM((CHUNK, D_EDGE), jnp.float32),
        pltpu.VMEM((CHUNK * D_EDGE // 128, 128), jnp.float32),
        pltpu.VMEM_SHARED((N_PAD, D_FEAT), jnp.float32),
        pltpu.VMEM_SHARED((N_PAD, D_EDGE), jnp.float32),
        pltpu.SemaphoreType.DMA,
    ],
)(_sc_segsum_kernel)


def _mm_body(sx_ref, se_ref, w1_ref, w2_ref, o_ref):
    sx = sx_ref[0] + sx_ref[1]
    se = se_ref[0]
    o_ref[...] = (
        jnp.dot(sx, w1_ref[...], preferred_element_type=jnp.float32)
        + jnp.dot(se, w2_ref[...], preferred_element_type=jnp.float32)
    )


_BR = 400


def _tc_matmul(sx, se, w1, w2):
    return pl.pallas_call(
        _mm_body,
        grid=(N_NODES // _BR,),
        in_specs=[
            pl.BlockSpec((NC, _BR, D_FEAT), lambda i: (0, i, 0)),
            pl.BlockSpec((1, _BR, D_EDGE), lambda i: (0, i, 0)),
            pl.BlockSpec((D_FEAT, D_OUT), lambda i: (0, 0)),
            pl.BlockSpec((D_EDGE, D_OUT), lambda i: (0, 0)),
        ],
        out_specs=pl.BlockSpec((_BR, D_OUT), lambda i: (i, 0)),
        out_shape=jax.ShapeDtypeStruct((N_NODES, D_OUT), jnp.float32),
    )(sx, se, w1, w2)


def kernel(node_feature, edge_index, edge_feature, W):
    src = edge_index[0].astype(jnp.int32).reshape(NW, NGRP, GRP, CHUNK)
    dst = edge_index[1].astype(jnp.int32).reshape(NW, NGRP, GRP, CHUNK)
    ef = edge_feature.reshape(NW, NGRP, GRP, CHUNK * D_EDGE // 128, 128)
    ridx = ((jnp.arange(NS)[:, None] + NS * jnp.arange(NROWCH // NS)[None, :])
            .astype(jnp.int32) * CHUNK)[:, :, None] + jnp.arange(
                CHUNK, dtype=jnp.int32)[None, None, :]
    sx, _ = _sc_segsum(node_feature, src, dst, ef, ridx)
    sx = sx.reshape(NC, N_PAD, D_FEAT)
    se = jax.ops.segment_sum(edge_feature, edge_index[1], num_segments=N_NODES)
    return _tc_matmul(sx, se[None], W[:D_FEAT], W[D_FEAT:])
